# baseline (speedup 1.0000x reference)
import jax
import jax.numpy as jnp
from jax import lax
from jax.experimental import pallas as pl
from jax.experimental.pallas import tpu as pltpu
from jax.experimental.pallas import tpu_sc as plsc

NREL = 200
ENT_DIM = 128
REL_DIM = 16
IN_CH = ENT_DIM + REL_DIM
NBASES = 4
N_TRI = 50000
AFLAT = NBASES * IN_CH

NC, NS, L = 2, 16, 16
NW = NC * NS
CHUNK = 1568
NBLK = CHUNK // L
CAP = 2 * CHUNK + L


def _splat(vec, j):
    idx = jnp.full((L,), j, dtype=jnp.int32)
    return vec.at[idx].get(mode="promise_in_bounds")


def _sc_body(u_hbm, trip_hbm, att_hbm, relemb_hbm, ent_hbm,
             parts_hbm, cnt_hbm,
             trip_ref, u_ref, att_ref, relemb_ref,
             gidx_ref, ridx_ref, c0_ref, c1_ref, c2_ref, c3_ref,
             rows_ref, stage_ref, cstage_ref, sem):
    wid = lax.axis_index("s") * NC + lax.axis_index("c")
    base = jnp.minimum(wid * CHUNK, N_TRI - CHUNK)
    skip = wid * CHUNK - base
    pltpu.sync_copy(trip_hbm.at[pl.ds(base * 3, CHUNK * 3)], trip_ref)
    pltpu.sync_copy(att_hbm, att_ref)
    pltpu.sync_copy(relemb_hbm, relemb_ref)
    pltpu.sync_copy(u_hbm, u_ref)

    uv = u_ref[...]
    iota = lax.broadcasted_iota(jnp.int32, (L,), 0)
    crefs = (c0_ref, c1_ref, c2_ref, c3_ref)

    def scan_blk(b, nv):
        lidx = b * L + iota
        sv = plsc.load_gather(trip_ref, [lidx * 3])
        dv = plsc.load_gather(trip_ref, [lidx * 3 + 2])
        mv = lidx >= skip
        m1 = (dv == uv) & mv
        m2 = (sv == uv) & mv

        def do(nv):
            rv = plsc.load_gather(trip_ref, [lidx * 3 + 1])

            def emit(nbase, mask, ent_idx, att_row):
                mi = mask.astype(jnp.int32)
                incl = plsc.cumsum(mi)
                pos = nbase + incl - mi
                plsc.store_scatter(gidx_ref, [pos], ent_idx, mask=mask)
                plsc.store_scatter(ridx_ref, [pos], rv, mask=mask)
                for bb in range(NBASES):
                    cb = plsc.load_gather(att_ref, [att_row * NBASES + bb])
                    plsc.store_scatter(crefs[bb], [pos], cb, mask=mask)
                return nbase + _splat(incl, L - 1)

            nv = emit(nv, m1, sv, rv)
            nv = emit(nv, m2, dv, rv + NREL)
            return nv

        return lax.cond(jnp.any(m1 | m2), do, lambda n: n, nv)

    nv = lax.fori_loop(0, NBLK, scan_blk, jnp.zeros((L,), jnp.int32))
    n = jnp.max(nv)

    zpos = nv + iota
    zi = jnp.zeros((L,), jnp.int32)
    zf = jnp.zeros((L,), jnp.float32)
    plsc.store_scatter(gidx_ref, [zpos], zi)
    plsc.store_scatter(ridx_ref, [zpos], zi)
    for cref in crefs:
        plsc.store_scatter(cref, [zpos], zf)

    nblocks = (n + L - 1) // L

    def match_blk(jb, accs):
        o = jb * L
        idxv = gidx_ref[pl.ds(o, L)]
        cp = pltpu.async_copy(ent_hbm.at[idxv], rows_ref, sem)
        rix = ridx_ref[pl.ds(o, L)]
        cs = [cref[pl.ds(o, L)] for cref in crefs]
        cp.wait()
        accs = list(accs)
        for j in range(L):
            csp = [_splat(cs[bb], j) for bb in range(NBASES)]
            rsp = _splat(rix, j)
            relvec = plsc.load_gather(relemb_ref, [rsp * REL_DIM + iota])
            for k in range(ENT_DIM // L):
                fv = rows_ref[j, pl.ds(k * L, L)]
                for bb in range(NBASES):
                    accs[bb * 9 + k] = accs[bb * 9 + k] + csp[bb] * fv
            for bb in range(NBASES):
                accs[bb * 9 + 8] = accs[bb * 9 + 8] + csp[bb] * relvec
        return tuple(accs)

    acc0 = tuple(zf for _ in range(NBASES * 9))
    accs = lax.fori_loop(0, nblocks, match_blk, acc0)

    for bb in range(NBASES):
        for k in range(9):
            stage_ref[pl.ds(bb * IN_CH + k * L, L)] = accs[bb * 9 + k]
    pltpu.sync_copy(stage_ref, parts_hbm.at[wid])
    cstage_ref[...] = jnp.where(iota == 0, nv.astype(jnp.float32), zf)
    pltpu.sync_copy(cstage_ref, cnt_hbm.at[wid])


def _tc_body(part_ref, cnt_ref, basis_ref, out_ref):
    prod = jnp.dot(part_ref[...], basis_ref[...],
                   preferred_element_type=jnp.float32)
    s = jnp.sum(prod, axis=0, keepdims=True)
    cnt = jnp.sum(cnt_ref[...])
    out_ref[...] = jnp.maximum(s / jnp.maximum(cnt, 1.0), 0.0)


@jax.jit
def kernel(unseen_entity, triplets, entity_table, relation_embedding, basis,
           att):
    trip = jnp.asarray(triplets).astype(jnp.int32).reshape(-1)
    u_arr = jnp.full((L,), jnp.asarray(unseen_entity, jnp.int32))
    att_flat = att.reshape(-1)
    relemb_flat = relation_embedding.reshape(-1)

    sc_fn = pl.kernel(
        _sc_body,
        out_type=(jax.ShapeDtypeStruct((NW, AFLAT), jnp.float32),
                  jax.ShapeDtypeStruct((NW, L), jnp.float32)),
        mesh=plsc.VectorSubcoreMesh(core_axis_name="c", subcore_axis_name="s",
                                    num_cores=NC, num_subcores=NS),
        compiler_params=pltpu.CompilerParams(needs_layout_passes=False),
        scratch_types=[
            pltpu.VMEM((CHUNK * 3,), jnp.int32),
            pltpu.VMEM((L,), jnp.int32),
            pltpu.VMEM((2 * NREL * NBASES,), jnp.float32),
            pltpu.VMEM((NREL * REL_DIM,), jnp.float32),
            pltpu.VMEM((CAP,), jnp.int32),
            pltpu.VMEM((CAP,), jnp.int32),
            pltpu.VMEM((CAP,), jnp.float32),
            pltpu.VMEM((CAP,), jnp.float32),
            pltpu.VMEM((CAP,), jnp.float32),
            pltpu.VMEM((CAP,), jnp.float32),
            pltpu.VMEM((L, ENT_DIM), jnp.float32),
            pltpu.VMEM((AFLAT,), jnp.float32),
            pltpu.VMEM((L,), jnp.float32),
            pltpu.SemaphoreType.DMA,
        ],
    )
    parts, counts = sc_fn(u_arr, trip, att_flat, relemb_flat, entity_table)

    out = pl.pallas_call(
        _tc_body,
        out_shape=jax.ShapeDtypeStruct((1, ENT_DIM), jnp.float32),
    )(parts, counts, basis.reshape(AFLAT, ENT_DIM))
    return out[0]


# speedup vs baseline: 2.0186x; 2.0186x over previous
import jax
import jax.numpy as jnp
from jax import lax
from jax.experimental import pallas as pl
from jax.experimental.pallas import tpu as pltpu
from jax.experimental.pallas import tpu_sc as plsc

NREL = 200
ENT_DIM = 128
REL_DIM = 16
IN_CH = ENT_DIM + REL_DIM
NBASES = 4
N_TRI = 50000
AFLAT = NBASES * IN_CH

NC, NS, L = 2, 16, 16
NW = NC * NS
CHUNK = 1600
NPAD = NW * CHUNK
SUB = 4
NBLK = CHUNK // (L * SUB)
CAP = 2 * CHUNK + L


def _splat(vec, j):
    idx = jnp.full((L,), j, dtype=jnp.int32)
    return vec.at[idx].get(mode="promise_in_bounds")


def _sc_body(u_hbm, src_hbm, dst_hbm, rel_hbm, att_hbm, relemb_hbm, ent_hbm,
             parts_hbm, cnt_hbm,
             src_ref, dst_ref, rel_ref, u_ref, att_ref, relemb_ref,
             gidx_ref, tidx_ref,
             rows_ref, stage_ref, cstage_ref, sem, sem2):
    wid = lax.axis_index("s") * NC + lax.axis_index("c")
    base = wid * CHUNK
    cp_s = pltpu.async_copy(src_hbm.at[pl.ds(base, CHUNK)], src_ref, sem)
    cp_d = pltpu.async_copy(dst_hbm.at[pl.ds(base, CHUNK)], dst_ref, sem)
    cp_r = pltpu.async_copy(rel_hbm.at[pl.ds(base, CHUNK)], rel_ref, sem)
    cp_u = pltpu.async_copy(u_hbm, u_ref, sem)
    cp_a = pltpu.async_copy(att_hbm, att_ref, sem2)
    cp_e = pltpu.async_copy(relemb_hbm, relemb_ref, sem2)
    cp_s.wait()
    cp_d.wait()
    cp_r.wait()
    cp_u.wait()

    uv = u_ref[...]
    iota = lax.broadcasted_iota(jnp.int32, (L,), 0)

    def scan_blk(b, nv):
        subs = []
        anym = None
        for s in range(SUB):
            off = (b * SUB + s) * L
            sv = src_ref[pl.ds(off, L)]
            dv = dst_ref[pl.ds(off, L)]
            m1 = dv == uv
            m2 = sv == uv
            m12 = m1 | m2
            anym = m12 if anym is None else (anym | m12)
            subs.append((off, sv, dv, m1, m2))

        def do(nv):
            def emit(nbase, mask, ent_idx, att_row):
                mi = mask.astype(jnp.int32)
                incl = plsc.cumsum(mi)
                pos = nbase + incl - mi
                plsc.store_scatter(gidx_ref, [pos], ent_idx, mask=mask)
                plsc.store_scatter(tidx_ref, [pos], att_row, mask=mask)
                return nbase + _splat(incl, L - 1)

            for off, sv, dv, m1, m2 in subs:
                rv = rel_ref[pl.ds(off, L)]
                nv = emit(nv, m1, sv, rv)
                nv = emit(nv, m2, dv, rv + NREL)
            return nv

        return lax.cond(jnp.any(anym), do, lambda n: n, nv)

    nv = lax.fori_loop(0, NBLK, scan_blk, jnp.zeros((L,), jnp.int32))
    n = jnp.max(nv)

    zpos = nv + iota
    zi = jnp.zeros((L,), jnp.int32)
    zf = jnp.zeros((L,), jnp.float32)
    plsc.store_scatter(gidx_ref, [zpos], zi)
    plsc.store_scatter(tidx_ref, [zpos], jnp.full((L,), -1, jnp.int32))

    cp_a.wait()
    cp_e.wait()

    nblocks = (n + L - 1) // L

    def match_blk(jb, accs):
        o = jb * L
        idxv = gidx_ref[pl.ds(o, L)]
        cp = pltpu.async_copy(ent_hbm.at[idxv], rows_ref, sem)
        tv = tidx_ref[pl.ds(o, L)]
        valid = tv >= 0
        tc = jnp.where(valid, tv, 0)
        riv = jnp.where(tc >= NREL, tc - NREL, tc)
        cs = []
        for bb in range(NBASES):
            cb = plsc.load_gather(att_ref, [tc * NBASES + bb])
            cs.append(jnp.where(valid, cb, 0.0))
        cp.wait()
        accs = list(accs)
        for j in range(L):
            csp = [_splat(cs[bb], j) for bb in range(NBASES)]
            rsp = _splat(riv, j)
            relvec = plsc.load_gather(relemb_ref, [rsp * REL_DIM + iota])
            for k in range(ENT_DIM // L):
                fv = rows_ref[j, pl.ds(k * L, L)]
                for bb in range(NBASES):
                    accs[bb * 9 + k] = accs[bb * 9 + k] + csp[bb] * fv
            for bb in range(NBASES):
                accs[bb * 9 + 8] = accs[bb * 9 + 8] + csp[bb] * relvec
        return tuple(accs)

    acc0 = tuple(zf for _ in range(NBASES * 9))
    accs = lax.fori_loop(0, nblocks, match_blk, acc0)

    for bb in range(NBASES):
        for k in range(9):
            stage_ref[pl.ds(bb * IN_CH + k * L, L)] = accs[bb * 9 + k]
    pltpu.sync_copy(stage_ref, parts_hbm.at[wid])
    cstage_ref[...] = jnp.where(iota == 0, nv.astype(jnp.float32), zf)
    pltpu.sync_copy(cstage_ref, cnt_hbm.at[wid])


def _tc_body(part_ref, cnt_ref, basis_ref, out_ref):
    prod = jnp.dot(part_ref[...], basis_ref[...],
                   preferred_element_type=jnp.float32,
                   precision=lax.Precision.HIGHEST)
    s = jnp.sum(prod, axis=0, keepdims=True)
    cnt = jnp.sum(cnt_ref[...])
    out_ref[...] = jnp.maximum(s / jnp.maximum(cnt, 1.0), 0.0)


@jax.jit
def kernel(unseen_entity, triplets, entity_table, relation_embedding, basis,
           att):
    trip = jnp.asarray(triplets).astype(jnp.int32)
    pad = NPAD - N_TRI
    src = jnp.concatenate([trip[:, 0], jnp.full((pad,), -1, jnp.int32)])
    rel = jnp.concatenate([trip[:, 1], jnp.zeros((pad,), jnp.int32)])
    dst = jnp.concatenate([trip[:, 2], jnp.full((pad,), -1, jnp.int32)])
    u_arr = jnp.full((L,), jnp.asarray(unseen_entity, jnp.int32))
    att_flat = att.reshape(-1)
    relemb_flat = relation_embedding.reshape(-1)

    sc_fn = pl.kernel(
        _sc_body,
        out_type=(jax.ShapeDtypeStruct((NW, AFLAT), jnp.float32),
                  jax.ShapeDtypeStruct((NW, L), jnp.float32)),
        mesh=plsc.VectorSubcoreMesh(core_axis_name="c", subcore_axis_name="s",
                                    num_cores=NC, num_subcores=NS),
        compiler_params=pltpu.CompilerParams(needs_layout_passes=False),
        scratch_types=[
            pltpu.VMEM((CHUNK,), jnp.int32),
            pltpu.VMEM((CHUNK,), jnp.int32),
            pltpu.VMEM((CHUNK,), jnp.int32),
            pltpu.VMEM((L,), jnp.int32),
            pltpu.VMEM((2 * NREL * NBASES,), jnp.float32),
            pltpu.VMEM((NREL * REL_DIM,), jnp.float32),
            pltpu.VMEM((CAP,), jnp.int32),
            pltpu.VMEM((CAP,), jnp.int32),
            pltpu.VMEM((L, ENT_DIM), jnp.float32),
            pltpu.VMEM((AFLAT,), jnp.float32),
            pltpu.VMEM((L,), jnp.float32),
            pltpu.SemaphoreType.DMA,
            pltpu.SemaphoreType.DMA,
        ],
    )
    parts, counts = sc_fn(u_arr, src, dst, rel, att_flat, relemb_flat,
                          entity_table)

    out = pl.pallas_call(
        _tc_body,
        out_shape=jax.ShapeDtypeStruct((1, ENT_DIM), jnp.float32),
    )(parts, counts, basis.reshape(AFLAT, ENT_DIM))
    return out[0]


# speedup vs baseline: 2.0877x; 1.0343x over previous
import jax
import jax.numpy as jnp
from jax import lax
from jax.experimental import pallas as pl
from jax.experimental.pallas import tpu as pltpu
from jax.experimental.pallas import tpu_sc as plsc

NREL = 200
ENT_DIM = 128
REL_DIM = 16
IN_CH = ENT_DIM + REL_DIM
NBASES = 4
N_TRI = 50000
AFLAT = NBASES * IN_CH

NC, NS, L = 2, 16, 16
NW = NC * NS
CHUNK = 1600
NPAD = NW * CHUNK
SUB = 4
NBLK = CHUNK // (L * SUB)
CAP = 2 * CHUNK + L


def _splat(vec, j):
    idx = jnp.full((L,), j, dtype=jnp.int32)
    return vec.at[idx].get(mode="promise_in_bounds")


def _sc_body(u_hbm, src_hbm, dst_hbm, rel_hbm, att_hbm, relemb_hbm, ent_hbm,
             parts_hbm, cnt_hbm,
             src_ref, dst_ref, rel_ref, u_ref, att_ref, relemb_ref,
             gidx_ref, tidx_ref,
             rows_ref, stage_ref, cstage_ref, sem, sem2):
    wid = lax.axis_index("s") * NC + lax.axis_index("c")
    base = jnp.minimum(wid * CHUNK, N_TRI - CHUNK)
    skip = wid * CHUNK - base
    cp_s = pltpu.async_copy(src_hbm.at[pl.ds(base, CHUNK)], src_ref, sem)
    cp_d = pltpu.async_copy(dst_hbm.at[pl.ds(base, CHUNK)], dst_ref, sem)
    cp_r = pltpu.async_copy(rel_hbm.at[pl.ds(base, CHUNK)], rel_ref, sem)
    cp_u = pltpu.async_copy(u_hbm, u_ref, sem)
    cp_a = pltpu.async_copy(att_hbm, att_ref, sem2)
    cp_e = pltpu.async_copy(relemb_hbm, relemb_ref, sem2)
    cp_s.wait()
    cp_d.wait()
    cp_r.wait()
    cp_u.wait()

    uv = u_ref[...]
    iota = lax.broadcasted_iota(jnp.int32, (L,), 0)

    def scan_blk(b, nv):
        subs = []
        anym = None
        for s in range(SUB):
            off = (b * SUB + s) * L
            sv = src_ref[pl.ds(off, L)]
            dv = dst_ref[pl.ds(off, L)]
            mv = iota >= (skip - off)
            m1 = (dv == uv) & mv
            m2 = (sv == uv) & mv
            m12 = m1 | m2
            anym = m12 if anym is None else (anym | m12)
            subs.append((off, sv, dv, m1, m2))

        def do(nv):
            def emit(nbase, mask, ent_idx, att_row):
                mi = mask.astype(jnp.int32)
                incl = plsc.cumsum(mi)
                pos = nbase + incl - mi
                plsc.store_scatter(gidx_ref, [pos], ent_idx, mask=mask)
                plsc.store_scatter(tidx_ref, [pos], att_row, mask=mask)
                return nbase + _splat(incl, L - 1)

            for off, sv, dv, m1, m2 in subs:
                rv = rel_ref[pl.ds(off, L)]
                nv = emit(nv, m1, sv, rv)
                nv = emit(nv, m2, dv, rv + NREL)
            return nv

        return lax.cond(jnp.any(anym), do, lambda n: n, nv)

    nv = lax.fori_loop(0, NBLK, scan_blk, jnp.zeros((L,), jnp.int32))
    n = jnp.max(nv)

    zpos = nv + iota
    zi = jnp.zeros((L,), jnp.int32)
    zf = jnp.zeros((L,), jnp.float32)
    plsc.store_scatter(gidx_ref, [zpos], zi)
    plsc.store_scatter(tidx_ref, [zpos], jnp.full((L,), -1, jnp.int32))

    cp_a.wait()
    cp_e.wait()

    nblocks = (n + L - 1) // L

    def match_blk(jb, accs):
        o = jb * L
        idxv = gidx_ref[pl.ds(o, L)]
        cp = pltpu.async_copy(ent_hbm.at[idxv], rows_ref, sem)
        tv = tidx_ref[pl.ds(o, L)]
        valid = tv >= 0
        tc = jnp.where(valid, tv, 0)
        riv = jnp.where(tc >= NREL, tc - NREL, tc)
        cs = []
        for bb in range(NBASES):
            cb = plsc.load_gather(att_ref, [tc + bb * (2 * NREL)])
            cs.append(jnp.where(valid, cb, 0.0))
        cp.wait()
        accs = list(accs)
        for j in range(L):
            csp = [_splat(cs[bb], j) for bb in range(NBASES)]
            rsp = _splat(riv, j)
            relvec = plsc.load_gather(relemb_ref, [iota * NREL + rsp])
            for k in range(ENT_DIM // L):
                fv = rows_ref[j, pl.ds(k * L, L)]
                for bb in range(NBASES):
                    accs[bb * 9 + k] = accs[bb * 9 + k] + csp[bb] * fv
            for bb in range(NBASES):
                accs[bb * 9 + 8] = accs[bb * 9 + 8] + csp[bb] * relvec
        return tuple(accs)

    acc0 = tuple(zf for _ in range(NBASES * 9))
    accs = lax.fori_loop(0, nblocks, match_blk, acc0)

    for bb in range(NBASES):
        for k in range(9):
            stage_ref[pl.ds(bb * IN_CH + k * L, L)] = accs[bb * 9 + k]
    pltpu.sync_copy(stage_ref, parts_hbm.at[wid])
    cstage_ref[...] = jnp.where(iota == 0, nv.astype(jnp.float32), zf)
    pltpu.sync_copy(cstage_ref, cnt_hbm.at[wid])


def _tc_body(part_ref, cnt_ref, basis_ref, out_ref):
    prod = jnp.dot(part_ref[...], basis_ref[...],
                   preferred_element_type=jnp.float32,
                   precision=lax.Precision.HIGHEST)
    s = jnp.sum(prod, axis=0, keepdims=True)
    cnt = jnp.sum(cnt_ref[...])
    out_ref[...] = jnp.maximum(s / jnp.maximum(cnt, 1.0), 0.0)


@jax.jit
def kernel(unseen_entity, triplets, entity_table, relation_embedding, basis,
           att):
    trip = jnp.asarray(triplets).astype(jnp.int32)
    src = trip[:, 0]
    rel = trip[:, 1]
    dst = trip[:, 2]
    u_arr = jnp.full((L,), jnp.asarray(unseen_entity, jnp.int32))
    att_flat = att.T.reshape(-1)
    relemb_flat = relation_embedding.T.reshape(-1)

    sc_fn = pl.kernel(
        _sc_body,
        out_type=(jax.ShapeDtypeStruct((NW, AFLAT), jnp.float32),
                  jax.ShapeDtypeStruct((NW, L), jnp.float32)),
        mesh=plsc.VectorSubcoreMesh(core_axis_name="c", subcore_axis_name="s",
                                    num_cores=NC, num_subcores=NS),
        compiler_params=pltpu.CompilerParams(needs_layout_passes=False),
        scratch_types=[
            pltpu.VMEM((CHUNK,), jnp.int32),
            pltpu.VMEM((CHUNK,), jnp.int32),
            pltpu.VMEM((CHUNK,), jnp.int32),
            pltpu.VMEM((L,), jnp.int32),
            pltpu.VMEM((2 * NREL * NBASES,), jnp.float32),
            pltpu.VMEM((NREL * REL_DIM,), jnp.float32),
            pltpu.VMEM((CAP,), jnp.int32),
            pltpu.VMEM((CAP,), jnp.int32),
            pltpu.VMEM((L, ENT_DIM), jnp.float32),
            pltpu.VMEM((AFLAT,), jnp.float32),
            pltpu.VMEM((L,), jnp.float32),
            pltpu.SemaphoreType.DMA,
            pltpu.SemaphoreType.DMA,
        ],
    )
    parts, counts = sc_fn(u_arr, src, dst, rel, att_flat, relemb_flat,
                          entity_table)

    out = pl.pallas_call(
        _tc_body,
        out_shape=jax.ShapeDtypeStruct((1, ENT_DIM), jnp.float32),
    )(parts, counts, basis.reshape(AFLAT, ENT_DIM))
    return out[0]


# speedup vs baseline: 2.1610x; 1.0351x over previous
import jax
import jax.numpy as jnp
from jax import lax
from jax.experimental import pallas as pl
from jax.experimental.pallas import tpu as pltpu
from jax.experimental.pallas import tpu_sc as plsc

NREL = 200
ENT_DIM = 128
REL_DIM = 16
IN_CH = ENT_DIM + REL_DIM
NBASES = 4
N_TRI = 50000
AFLAT = NBASES * IN_CH

NC, NS, L = 2, 16, 16
NW = NC * NS
CHUNK = 1600
NPAD = NW * CHUNK
SUB = 4
NBLK = CHUNK // (L * SUB)
CAP = 2 * CHUNK + L


def _splat(vec, j):
    idx = jnp.full((L,), j, dtype=jnp.int32)
    return vec.at[idx].get(mode="promise_in_bounds")


ATT_OFF = L
REL_OFF = L + NBASES * 2 * NREL
TAB_LEN = REL_OFF + REL_DIM * NREL


def _sc_body(tab_hbm, src_hbm, dst_hbm, rel_hbm, ent_hbm,
             parts_hbm, cnt_hbm,
             src_ref, dst_ref, rel_ref, u_ref, tab_ref,
             gidx_ref, tidx_ref,
             rows_ref, stage_ref, cstage_ref, sem, sem2):
    wid = lax.axis_index("s") * NC + lax.axis_index("c")
    base = jnp.minimum(wid * CHUNK, N_TRI - CHUNK)
    skip = wid * CHUNK - base
    cp_s = pltpu.async_copy(src_hbm.at[pl.ds(base, CHUNK)], src_ref, sem)
    cp_d = pltpu.async_copy(dst_hbm.at[pl.ds(base, CHUNK)], dst_ref, sem)
    cp_r = pltpu.async_copy(rel_hbm.at[pl.ds(base, CHUNK)], rel_ref, sem)
    cp_u = pltpu.async_copy(tab_hbm.at[pl.ds(0, L)], u_ref, sem)
    cp_a = pltpu.async_copy(tab_hbm, tab_ref, sem2)
    cp_s.wait()
    cp_d.wait()
    cp_r.wait()
    cp_u.wait()

    uv = plsc.bitcast(u_ref[...], jnp.int32)
    iota = lax.broadcasted_iota(jnp.int32, (L,), 0)

    def scan_blk(b, nv):
        subs = []
        anym = None
        for s in range(SUB):
            off = (b * SUB + s) * L
            sv = src_ref[pl.ds(off, L)]
            dv = dst_ref[pl.ds(off, L)]
            mv = iota >= (skip - off)
            m1 = (dv == uv) & mv
            m2 = (sv == uv) & mv
            m12 = m1 | m2
            anym = m12 if anym is None else (anym | m12)
            subs.append((off, sv, dv, m1, m2))

        def do(nv):
            def emit(nbase, mask, ent_idx, att_row):
                mi = mask.astype(jnp.int32)
                incl = plsc.cumsum(mi)
                pos = nbase + incl - mi
                plsc.store_scatter(gidx_ref, [pos], ent_idx, mask=mask)
                plsc.store_scatter(tidx_ref, [pos], att_row, mask=mask)
                return nbase + _splat(incl, L - 1)

            for off, sv, dv, m1, m2 in subs:
                rv = rel_ref[pl.ds(off, L)]
                nv = emit(nv, m1, sv, rv)
                nv = emit(nv, m2, dv, rv + NREL)
            return nv

        return lax.cond(jnp.any(anym), do, lambda n: n, nv)

    nv = lax.fori_loop(0, NBLK, scan_blk, jnp.zeros((L,), jnp.int32))
    n = jnp.max(nv)

    zpos = nv + iota
    zi = jnp.zeros((L,), jnp.int32)
    zf = jnp.zeros((L,), jnp.float32)
    plsc.store_scatter(gidx_ref, [zpos], zi)
    plsc.store_scatter(tidx_ref, [zpos], jnp.full((L,), -1, jnp.int32))

    cp_a.wait()

    nblocks = (n + L - 1) // L

    def match_blk(jb, accs):
        o = jb * L
        idxv = gidx_ref[pl.ds(o, L)]
        cp = pltpu.async_copy(ent_hbm.at[idxv], rows_ref, sem)
        tv = tidx_ref[pl.ds(o, L)]
        valid = tv >= 0
        tc = jnp.where(valid, tv, 0)
        riv = jnp.where(tc >= NREL, tc - NREL, tc)
        cs = []
        for bb in range(NBASES):
            cb = plsc.load_gather(tab_ref, [tc + (ATT_OFF + bb * 2 * NREL)])
            cs.append(jnp.where(valid, cb, 0.0))
        cp.wait()
        accs = list(accs)
        for j in range(L):
            csp = [_splat(cs[bb], j) for bb in range(NBASES)]
            rsp = _splat(riv, j)
            relvec = plsc.load_gather(tab_ref, [iota * NREL + (rsp + REL_OFF)])
            for k in range(ENT_DIM // L):
                fv = rows_ref[j, pl.ds(k * L, L)]
                for bb in range(NBASES):
                    accs[bb * 9 + k] = accs[bb * 9 + k] + csp[bb] * fv
            for bb in range(NBASES):
                accs[bb * 9 + 8] = accs[bb * 9 + 8] + csp[bb] * relvec
        return tuple(accs)

    acc0 = tuple(zf for _ in range(NBASES * 9))
    accs = lax.fori_loop(0, nblocks, match_blk, acc0)

    for bb in range(NBASES):
        for k in range(9):
            stage_ref[pl.ds(bb * IN_CH + k * L, L)] = accs[bb * 9 + k]
    pltpu.sync_copy(stage_ref, parts_hbm.at[wid])
    cstage_ref[...] = jnp.where(iota == 0, nv.astype(jnp.float32), zf)
    pltpu.sync_copy(cstage_ref, cnt_hbm.at[wid])


def _tc_body(part_ref, cnt_ref, basis_ref, out_ref):
    prod = jnp.dot(part_ref[...], basis_ref[...],
                   preferred_element_type=jnp.float32,
                   precision=lax.Precision.HIGHEST)
    s = jnp.sum(prod, axis=0, keepdims=True)
    cnt = jnp.sum(cnt_ref[...])
    out_ref[...] = jnp.maximum(s / jnp.maximum(cnt, 1.0), 0.0)


@jax.jit
def kernel(unseen_entity, triplets, entity_table, relation_embedding, basis,
           att):
    trip = jnp.asarray(triplets).astype(jnp.int32)
    src = trip[:, 0]
    rel = trip[:, 1]
    dst = trip[:, 2]
    u_arr = jnp.full((L,), jnp.asarray(unseen_entity, jnp.int32))
    tab = jnp.concatenate([
        lax.bitcast_convert_type(u_arr, jnp.float32),
        att.T.reshape(-1),
        relation_embedding.T.reshape(-1),
    ])

    sc_fn = pl.kernel(
        _sc_body,
        out_type=(jax.ShapeDtypeStruct((NW, AFLAT), jnp.float32),
                  jax.ShapeDtypeStruct((NW, L), jnp.float32)),
        mesh=plsc.VectorSubcoreMesh(core_axis_name="c", subcore_axis_name="s",
                                    num_cores=NC, num_subcores=NS),
        compiler_params=pltpu.CompilerParams(needs_layout_passes=False),
        scratch_types=[
            pltpu.VMEM((CHUNK,), jnp.int32),
            pltpu.VMEM((CHUNK,), jnp.int32),
            pltpu.VMEM((CHUNK,), jnp.int32),
            pltpu.VMEM((L,), jnp.float32),
            pltpu.VMEM((TAB_LEN,), jnp.float32),
            pltpu.VMEM((CAP,), jnp.int32),
            pltpu.VMEM((CAP,), jnp.int32),
            pltpu.VMEM((L, ENT_DIM), jnp.float32),
            pltpu.VMEM((AFLAT,), jnp.float32),
            pltpu.VMEM((L,), jnp.float32),
            pltpu.SemaphoreType.DMA,
            pltpu.SemaphoreType.DMA,
        ],
    )
    parts, counts = sc_fn(tab, src, dst, rel, entity_table)

    out = pl.pallas_call(
        _tc_body,
        out_shape=jax.ShapeDtypeStruct((1, ENT_DIM), jnp.float32),
    )(parts, counts, basis.reshape(AFLAT, ENT_DIM))
    return out[0]
